# Initial kernel scaffold; baseline (speedup 1.0000x reference)
import jax
import jax.numpy as jnp
from jax.experimental import pallas as pl


def kernel(x, embedding_weight):
    raise NotImplementedError("write your pallas kernel here")


# speedup vs baseline: 1.4763x; 1.4763x over previous
import functools

import jax
import jax.numpy as jnp
from jax import lax
from jax.experimental import pallas as pl
from jax.experimental.pallas import tpu as pltpu
from jax.experimental.pallas import tpu_sc as plsc

VOCAB_SIZE = 1000000
EMBED_DIM = 32
BATCH = 4096
SEQ_LEN = 200

_B = BATCH * SEQ_LEN

_info = plsc.get_sparse_core_info()
_NC, _NS = _info.num_cores, _info.num_subcores
_NW = _NC * _NS
_B_PER_W = _B // _NW
_CHUNK = 1600
_N_CHUNKS = _B_PER_W // _CHUNK


@functools.partial(
    pl.kernel,
    mesh=plsc.VectorSubcoreMesh(core_axis_name="c", subcore_axis_name="s"),
    out_type=jax.ShapeDtypeStruct((_B, EMBED_DIM), jnp.float32),
    scratch_types=[
        pltpu.VMEM((_CHUNK,), jnp.int32),
        pltpu.VMEM((_CHUNK, EMBED_DIM), jnp.float32),
        pltpu.SemaphoreType.DMA,
    ],
    compiler_params=pltpu.CompilerParams(use_tc_tiling_on_sc=False),
)
def _sc_gather(table_hbm, idx_hbm, out_hbm, idx_v, rows_v, sem):
    wid = lax.axis_index("s") * _NC + lax.axis_index("c")
    w_base = wid * _B_PER_W

    def chunk_body(i, carry):
        base = w_base + i * _CHUNK
        pltpu.sync_copy(idx_hbm.at[pl.ds(base, _CHUNK)], idx_v)
        pltpu.async_copy(table_hbm.at[idx_v], rows_v, sem).wait()
        pltpu.sync_copy(rows_v, out_hbm.at[pl.ds(base, _CHUNK)])
        return carry

    lax.fori_loop(0, _N_CHUNKS, chunk_body, 0)


def kernel(x, embedding_weight):
    idx = x.reshape(_B)
    out = _sc_gather(embedding_weight, idx)
    return out.reshape(BATCH, SEQ_LEN, EMBED_DIM)


# speedup vs baseline: 1.4881x; 1.0079x over previous
import functools

import jax
import jax.numpy as jnp
from jax import lax
from jax.experimental import pallas as pl
from jax.experimental.pallas import tpu as pltpu
from jax.experimental.pallas import tpu_sc as plsc

VOCAB_SIZE = 1000000
EMBED_DIM = 32
BATCH = 4096
SEQ_LEN = 200

_B = BATCH * SEQ_LEN

_info = plsc.get_sparse_core_info()
_NC, _NS = _info.num_cores, _info.num_subcores
_NW = _NC * _NS
_B_PER_W = _B // _NW
_ROWS_PER_CHUNK = 8
_CHUNK = _ROWS_PER_CHUNK * SEQ_LEN
_N_CHUNKS = _B_PER_W // _CHUNK


@functools.partial(
    pl.kernel,
    mesh=plsc.VectorSubcoreMesh(core_axis_name="c", subcore_axis_name="s"),
    out_type=jax.ShapeDtypeStruct((BATCH, SEQ_LEN, EMBED_DIM), jnp.float32),
    scratch_types=[
        pltpu.VMEM((_CHUNK,), jnp.int32),
        pltpu.VMEM((_CHUNK,), jnp.int32),
        pltpu.VMEM((_CHUNK, EMBED_DIM), jnp.float32),
        pltpu.VMEM((_CHUNK, EMBED_DIM), jnp.float32),
        pltpu.SemaphoreType.DMA,
        pltpu.SemaphoreType.DMA,
        pltpu.SemaphoreType.DMA,
        pltpu.SemaphoreType.DMA,
    ],
    compiler_params=pltpu.CompilerParams(use_tc_tiling_on_sc=False),
)
def _sc_gather(
    table_hbm, idx_hbm, out_hbm, idx_v0, idx_v1, rows_v0, rows_v1,
    gsem0, gsem1, wsem0, wsem1,
):
    wid = lax.axis_index("s") * _NC + lax.axis_index("c")
    w_base = wid * _B_PER_W
    w_brow = wid * (_B_PER_W // SEQ_LEN)

    idx_bufs = (idx_v0, idx_v1)
    row_bufs = (rows_v0, rows_v1)
    gsems = (gsem0, gsem1)
    wsems = (wsem0, wsem1)
    writes = [[], []]
    for i in range(_N_CHUNKS):
        b = i % 2
        for cp in writes[b]:
            cp.wait()
        writes[b] = []
        pltpu.sync_copy(
            idx_hbm.at[pl.ds(w_base + i * _CHUNK, _CHUNK)], idx_bufs[b]
        )
        pltpu.async_copy(
            table_hbm.at[idx_bufs[b]], row_bufs[b], gsems[b]
        ).wait()
        brow = w_brow + i * _ROWS_PER_CHUNK
        for j in range(_ROWS_PER_CHUNK):
            cp = pltpu.async_copy(
                row_bufs[b].at[pl.ds(j * SEQ_LEN, SEQ_LEN)],
                out_hbm.at[brow + j],
                wsems[b],
            )
            writes[b].append(cp)
    for b in (0, 1):
        for cp in writes[b]:
            cp.wait()


def kernel(x, embedding_weight):
    idx = x.reshape(_B)
    return _sc_gather(embedding_weight, idx)
